# baseline (speedup 1.0000x reference)
import functools

import jax
import jax.numpy as jnp
from jax import lax
from jax.experimental import pallas as pl
from jax.experimental.pallas import tpu as pltpu
from jax.experimental.pallas import tpu_sc as plsc

N = 10000
E = 320000
F = 128
G = 64

NC = 2
NS = 16
NW = NC * NS
NH = 1
CH = 128
NCH = 80
PER_W = NCH * CH
EH = NW * PER_W
EPAD = NH * EH
NBUF = 2
NPAD = 10240
SEG = NPAD // NS

BE = 4096
NBLK = EH // BE


def _pack_halves(m):
    lo = lax.bitcast_convert_type(
        m[:, :F].astype(jnp.bfloat16).astype(jnp.float32), jnp.uint32)
    hi = lax.bitcast_convert_type(
        m[:, F:].astype(jnp.bfloat16).astype(jnp.float32), jnp.uint32)
    word = (lo >> 16) | (hi & jnp.uint32(0xFFFF0000))
    return lax.bitcast_convert_type(word, jnp.int32)


def _unpack_halves(w):
    u = lax.bitcast_convert_type(w, jnp.uint32)
    lo = lax.bitcast_convert_type(u << 16, jnp.float32)
    hi = lax.bitcast_convert_type(u & jnp.uint32(0xFFFF0000), jnp.float32)
    return lo, hi


def _tc_pre_body(x_ref, w0_ref, b0_ref, gw_ref, batch_ref, a_ref, b_ref,
                 xgseg_ref, cnt_ref):
    x = x_ref[...]
    w0 = w0_ref[...]
    a = jnp.dot(x, w0[:F, :], preferred_element_type=jnp.float32)
    b = jnp.dot(x, w0[F:, :], preferred_element_type=jnp.float32) \
        + b0_ref[...][None, :]
    a_ref[...] = _pack_halves(a)
    b_ref[...] = _pack_halves(b)
    ids = lax.broadcasted_iota(jnp.int32, (G, N), 0)
    maskf = (ids == batch_ref[...][None, :]).astype(jnp.float32)
    cnt_ref[...] = jnp.sum(maskf, axis=1, keepdims=True)
    sumx = jnp.dot(maskf, x, preferred_element_type=jnp.float32)
    xgseg_ref[...] = jnp.dot(sumx, gw_ref[...][:F, :],
                             preferred_element_type=jnp.float32)


def _tc_pre(x, phi_W0, phi_b0, gamma_W, batch32):
    return pl.pallas_call(
        _tc_pre_body,
        out_shape=(
            jax.ShapeDtypeStruct((N, F), jnp.int32),
            jax.ShapeDtypeStruct((N, F), jnp.int32),
            jax.ShapeDtypeStruct((G, 1), jnp.float32),
            jax.ShapeDtypeStruct((G, 1), jnp.float32),
        ),
    )(x, phi_W0, phi_b0, gamma_W, batch32)


def _sc_gather_body(a_hbm, b_hbm, sd_hbm, pa_hbm, pb_hbm,
                    ipk, idxa, idxb, bufa, bufb, sems):
    wid = lax.axis_index("s") * NC + lax.axis_index("c")
    base0 = wid * PER_W
    pltpu.sync_copy(sd_hbm.at[wid], ipk)

    def unpack(c, b):
        for j in range(CH // 16):
            w = ipk[c, pl.ds(j * 16, 16)]
            idxa[b][pl.ds(j * 16, 16)] = w & 0xFFFF
            idxb[b][pl.ds(j * 16, 16)] = (w >> 16) & 0xFFFF

    def issue(c, b):
        pltpu.async_copy(a_hbm.at[idxa[b]], bufa[b], sems[b])
        pltpu.async_copy(b_hbm.at[idxb[b]], bufb[b], sems[b])

    for b in range(NBUF):
        unpack(b, b)
        issue(b, b)

    def outer(g, _):
        for b in range(NBUF):
            c = g * NBUF + b
            pltpu.make_async_copy(a_hbm.at[idxa[b]], bufa[b], sems[b]).wait()
            pltpu.make_async_copy(b_hbm.at[idxb[b]], bufb[b], sems[b]).wait()
            base = base0 + c * CH
            pltpu.sync_copy(bufa[b], pa_hbm.at[pl.ds(base, CH)])
            pltpu.sync_copy(bufb[b], pb_hbm.at[pl.ds(base, CH)])

            @pl.when(c + NBUF < NCH)
            def _():
                unpack(c + NBUF, b)
                issue(c + NBUF, b)
        return 0

    lax.fori_loop(0, NCH // NBUF, outer, 0)


def _sc_gather(A, B, sd3):
    mesh = plsc.VectorSubcoreMesh(core_axis_name="c", subcore_axis_name="s")
    f = pl.kernel(
        _sc_gather_body,
        out_type=(
            jax.ShapeDtypeStruct((EH, F), jnp.int32),
            jax.ShapeDtypeStruct((EH, F), jnp.int32),
        ),
        mesh=mesh,
        scratch_types=(
            pltpu.VMEM((NCH, CH), jnp.int32),
            [pltpu.VMEM((CH,), jnp.int32) for _ in range(NBUF)],
            [pltpu.VMEM((CH,), jnp.int32) for _ in range(NBUF)],
            [pltpu.VMEM((CH, F), jnp.int32) for _ in range(NBUF)],
            [pltpu.VMEM((CH, F), jnp.int32) for _ in range(NBUF)],
            [pltpu.SemaphoreType.DMA for _ in range(NBUF)],
        ),
    )
    return f(A, B, sd3)


def _tc_mlp_body(pa_ref, pb_ref, w1_ref, b1_ref, wout_ref, gwt_ref, t_ref):
    pa_lo, pa_hi = _unpack_halves(pa_ref[...])
    pb_lo, pb_hi = _unpack_halves(pb_ref[...])
    h_lo = jnp.tanh(pa_lo + pb_lo).astype(jnp.bfloat16)
    h_hi = jnp.tanh(pa_hi + pb_hi).astype(jnp.bfloat16)
    w1 = w1_ref[...]
    z = (jnp.dot(h_lo, w1[:F, :], preferred_element_type=jnp.float32)
         + jnp.dot(h_hi, w1[F:, :], preferred_element_type=jnp.float32)
         + b1_ref[...][None, :])
    h1 = jnp.tanh(z).astype(jnp.bfloat16)
    msg = jnp.tanh(jnp.dot(h1, wout_ref[...],
                           preferred_element_type=jnp.float32))
    gw2_row = gwt_ref[...][:, F:]
    t_row = lax.dot_general(gw2_row, msg, (((1,), (1,)), ((), ())),
                            preferred_element_type=jnp.float32)
    t_ref[...] = t_row.reshape(1, 1, BE)


def _tc_mlp(PA, PB, phi_W1, phi_b1, phi_Wout, gamma_W):
    return pl.pallas_call(
        _tc_mlp_body,
        grid=(NBLK,),
        in_specs=[
            pl.BlockSpec((BE, F), lambda i: (i, 0)),
            pl.BlockSpec((BE, F), lambda i: (i, 0)),
            pl.BlockSpec((2 * F, 2 * F), lambda i: (0, 0)),
            pl.BlockSpec((2 * F,), lambda i: (0,)),
            pl.BlockSpec((2 * F, F), lambda i: (0, 0)),
            pl.BlockSpec((1, 2 * F), lambda i: (0, 0)),
        ],
        out_specs=pl.BlockSpec((1, 1, BE), lambda i: (i, 0, 0)),
        out_shape=jax.ShapeDtypeStruct((NBLK, 1, BE), jnp.float32),
        name="tc_mlp",
        compiler_params=pltpu.CompilerParams(
            dimension_semantics=("arbitrary",)),
    )(PA, PB, phi_W1.astype(jnp.bfloat16), phi_b1,
      phi_Wout.astype(jnp.bfloat16), gamma_W.reshape(1, 2 * F))


def _sc_scatter_body(t_hbm, src_hbm, out_hbm, acc, tbuf, ibuf, zbuf, sem):
    cid = lax.axis_index("c")
    sid = lax.axis_index("s")
    wid = sid * NC + cid

    def zstep(j, _):
        zbuf[pl.ds(j * 16, 16)] = jnp.zeros((16,), jnp.float32)
        return 0

    lax.fori_loop(0, SEG // 16, zstep, 0)
    pltpu.sync_copy(zbuf, acc.at[pl.ds(sid * SEG, SEG)])
    pltpu.sync_copy(t_hbm.at[wid], tbuf)
    pltpu.sync_copy(src_hbm.at[wid], ibuf)
    plsc.subcore_barrier()

    def step(g, _):
        for j in range(NBUF):
            r = g * NBUF + j
            pltpu.async_copy(tbuf.at[r], acc.at[ibuf.at[r]], sem, add=True)
        for j in range(NBUF):
            r = g * NBUF + j
            pltpu.make_async_copy(tbuf.at[r], acc.at[ibuf.at[r]], sem).wait()
        return 0

    lax.fori_loop(0, NCH // NBUF, step, 0)
    plsc.subcore_barrier()
    pltpu.sync_copy(acc.at[pl.ds(sid * SEG, SEG)],
                    out_hbm.at[pl.ds(cid * NPAD + sid * SEG, SEG)])


def _sc_scatter(t2, src2):
    mesh = plsc.VectorSubcoreMesh(core_axis_name="c", subcore_axis_name="s")
    f = pl.kernel(
        _sc_scatter_body,
        out_type=jax.ShapeDtypeStruct((NC * NPAD,), jnp.float32),
        mesh=mesh,
        scratch_types=(
            pltpu.VMEM_SHARED((NPAD,), jnp.float32),
            pltpu.VMEM((NCH, CH), jnp.float32),
            pltpu.VMEM((NCH, CH), jnp.int32),
            pltpu.VMEM((SEG,), jnp.float32),
            pltpu.SemaphoreType.DMA,
        ),
    )
    return f(t2, src2)


def _tc_final_body(p_ref, xgseg_ref, cnt_ref, batch_ref, gb_ref, out_ref):
    aggr = jnp.sum(p_ref[:, :N], axis=0, keepdims=True)
    ids = lax.broadcasted_iota(jnp.int32, (G, N), 0)
    maskf = (ids == batch_ref[...][None, :]).astype(jnp.float32)
    aggrseg = jnp.sum(maskf * aggr, axis=1, keepdims=True)
    cnt = cnt_ref[...]
    num = xgseg_ref[...] + aggrseg + gb_ref[...][None, :] * cnt
    out_ref[...] = num / jnp.maximum(cnt, 1.0)


def _tc_final(partials, xgseg, counts, batch32, gamma_b):
    return pl.pallas_call(
        _tc_final_body,
        out_shape=jax.ShapeDtypeStruct((G, 1), jnp.float32),
    )(partials, xgseg, counts, batch32, gamma_b)


def kernel(x, edge_index, batch, phi_W0, phi_b0, phi_W1, phi_b1, phi_Wout,
           gamma_W, gamma_b):
    src = edge_index[0].astype(jnp.int32)
    dst = edge_index[1].astype(jnp.int32)
    batch32 = batch.astype(jnp.int32)
    pad = EPAD - E
    sd4 = jnp.concatenate(
        [src | (dst << 16), jnp.zeros((pad,), jnp.int32)]
    ).reshape(NH, NW, NCH, CH)
    src4 = jnp.concatenate(
        [src, jnp.full((pad,), N, jnp.int32)]
    ).reshape(NH, NW, NCH, CH)
    A, B, xgseg, counts = _tc_pre(x, phi_W0, phi_b0, gamma_W, batch32)
    parts = []
    for h in range(NH):
        PA, PB = _sc_gather(A, B, sd4[h])
        t = _tc_mlp(PA, PB, phi_W1, phi_b1, phi_Wout, gamma_W)
        parts.append(_sc_scatter(t.reshape(NW, NCH, CH), src4[h]))
    partials = jnp.concatenate(parts).reshape(NH * NC, NPAD)
    return _tc_final(partials, xgseg, counts, batch32, gamma_b)


# speedup vs baseline: 1.9096x; 1.9096x over previous
import functools

import jax
import jax.numpy as jnp
from jax import lax
from jax.experimental import pallas as pl
from jax.experimental.pallas import tpu as pltpu
from jax.experimental.pallas import tpu_sc as plsc

N = 10000
E = 320000
F = 128
G = 64

NC = 2
NS = 16
NW = NC * NS
NH = 1
CH = 80
NCH = 125
PER_W = NCH * CH
EH = NW * PER_W
EPAD = NH * EH
NBUF = 5
NPAD = 10240
SEG = NPAD // NS

BE = 2560
NBLK = EH // BE


def _pack_halves(m):
    lo = lax.bitcast_convert_type(
        m[:, :F].astype(jnp.bfloat16).astype(jnp.float32), jnp.uint32)
    hi = lax.bitcast_convert_type(
        m[:, F:].astype(jnp.bfloat16).astype(jnp.float32), jnp.uint32)
    word = (lo >> 16) | (hi & jnp.uint32(0xFFFF0000))
    return lax.bitcast_convert_type(word, jnp.int32)


def _unpack_halves(w):
    u = lax.bitcast_convert_type(w, jnp.uint32)
    lo = lax.bitcast_convert_type(u << 16, jnp.float32)
    hi = lax.bitcast_convert_type(u & jnp.uint32(0xFFFF0000), jnp.float32)
    return lo, hi


def _tc_pre_body(x_ref, w0_ref, b0_ref, gw_ref, batch_ref, a_ref, b_ref,
                 xgseg_ref, cnt_ref):
    x = x_ref[...]
    w0 = w0_ref[...]
    a = jnp.dot(x, w0[:F, :], preferred_element_type=jnp.float32)
    b = jnp.dot(x, w0[F:, :], preferred_element_type=jnp.float32) \
        + b0_ref[...][None, :]
    a_ref[...] = _pack_halves(a)
    b_ref[...] = _pack_halves(b)
    ids = lax.broadcasted_iota(jnp.int32, (G, N), 0)
    maskf = (ids == batch_ref[...][None, :]).astype(jnp.float32)
    cnt_ref[...] = jnp.sum(maskf, axis=1, keepdims=True)
    sumx = jnp.dot(maskf, x, preferred_element_type=jnp.float32)
    xgseg_ref[...] = jnp.dot(sumx, gw_ref[...][:F, :],
                             preferred_element_type=jnp.float32)


def _tc_pre(x, phi_W0, phi_b0, gamma_W, batch32):
    return pl.pallas_call(
        _tc_pre_body,
        out_shape=(
            jax.ShapeDtypeStruct((N, F), jnp.int32),
            jax.ShapeDtypeStruct((N, F), jnp.int32),
            jax.ShapeDtypeStruct((G, 1), jnp.float32),
            jax.ShapeDtypeStruct((G, 1), jnp.float32),
        ),
    )(x, phi_W0, phi_b0, gamma_W, batch32)


def _sc_gather_body(a_hbm, b_hbm, sd_hbm, pa_hbm, pb_hbm,
                    ipk, idxa, idxb, bufa, bufb, sems):
    wid = lax.axis_index("s") * NC + lax.axis_index("c")
    base0 = wid * PER_W
    pltpu.sync_copy(sd_hbm.at[wid], ipk)

    def unpack(c, b):
        for j in range(CH // 16):
            w = ipk[c, pl.ds(j * 16, 16)]
            idxa[b][pl.ds(j * 16, 16)] = w & 0xFFFF
            idxb[b][pl.ds(j * 16, 16)] = (w >> 16) & 0xFFFF

    def issue(c, b):
        pltpu.async_copy(a_hbm.at[idxa[b]], bufa[b], sems[b])
        pltpu.async_copy(b_hbm.at[idxb[b]], bufb[b], sems[b])

    for b in range(NBUF):
        unpack(b, b)
        issue(b, b)

    def outer(g, _):
        for b in range(NBUF):
            c = g * NBUF + b
            pltpu.make_async_copy(a_hbm.at[idxa[b]], bufa[b], sems[b]).wait()
            pltpu.make_async_copy(b_hbm.at[idxb[b]], bufb[b], sems[b]).wait()
            base = base0 + c * CH
            pltpu.sync_copy(bufa[b], pa_hbm.at[pl.ds(base, CH)])
            pltpu.sync_copy(bufb[b], pb_hbm.at[pl.ds(base, CH)])

            @pl.when(c + NBUF < NCH)
            def _():
                unpack(c + NBUF, b)
                issue(c + NBUF, b)
        return 0

    lax.fori_loop(0, NCH // NBUF, outer, 0)


def _sc_gather(A, B, sd3):
    mesh = plsc.VectorSubcoreMesh(core_axis_name="c", subcore_axis_name="s")
    f = pl.kernel(
        _sc_gather_body,
        out_type=(
            jax.ShapeDtypeStruct((EH, F), jnp.int32),
            jax.ShapeDtypeStruct((EH, F), jnp.int32),
        ),
        mesh=mesh,
        scratch_types=(
            pltpu.VMEM((NCH, CH), jnp.int32),
            [pltpu.VMEM((CH,), jnp.int32) for _ in range(NBUF)],
            [pltpu.VMEM((CH,), jnp.int32) for _ in range(NBUF)],
            [pltpu.VMEM((CH, F), jnp.int32) for _ in range(NBUF)],
            [pltpu.VMEM((CH, F), jnp.int32) for _ in range(NBUF)],
            [pltpu.SemaphoreType.DMA for _ in range(NBUF)],
        ),
    )
    return f(A, B, sd3)


def _tc_mlp_body(pa_ref, pb_ref, w1_ref, b1_ref, wout_ref, gwt_ref, t_ref):
    pa_lo, pa_hi = _unpack_halves(pa_ref[...])
    pb_lo, pb_hi = _unpack_halves(pb_ref[...])
    h_lo = jnp.tanh(pa_lo + pb_lo).astype(jnp.bfloat16)
    h_hi = jnp.tanh(pa_hi + pb_hi).astype(jnp.bfloat16)
    w1 = w1_ref[...]
    z = (jnp.dot(h_lo, w1[:F, :], preferred_element_type=jnp.float32)
         + jnp.dot(h_hi, w1[F:, :], preferred_element_type=jnp.float32)
         + b1_ref[...][None, :])
    h1 = jnp.tanh(z).astype(jnp.bfloat16)
    msg = jnp.tanh(jnp.dot(h1, wout_ref[...],
                           preferred_element_type=jnp.float32))
    gw2_row = gwt_ref[...][:, F:]
    t_row = lax.dot_general(gw2_row, msg, (((1,), (1,)), ((), ())),
                            preferred_element_type=jnp.float32)
    t_ref[...] = t_row.reshape(1, 1, BE)


def _tc_mlp(PA, PB, phi_W1, phi_b1, phi_Wout, gamma_W):
    return pl.pallas_call(
        _tc_mlp_body,
        grid=(NBLK,),
        in_specs=[
            pl.BlockSpec((BE, F), lambda i: (i, 0)),
            pl.BlockSpec((BE, F), lambda i: (i, 0)),
            pl.BlockSpec((2 * F, 2 * F), lambda i: (0, 0)),
            pl.BlockSpec((2 * F,), lambda i: (0,)),
            pl.BlockSpec((2 * F, F), lambda i: (0, 0)),
            pl.BlockSpec((1, 2 * F), lambda i: (0, 0)),
        ],
        out_specs=pl.BlockSpec((1, 1, BE), lambda i: (i, 0, 0)),
        out_shape=jax.ShapeDtypeStruct((NBLK, 1, BE), jnp.float32),
        name="tc_mlp",
        compiler_params=pltpu.CompilerParams(
            dimension_semantics=("arbitrary",)),
    )(PA, PB, phi_W1.astype(jnp.bfloat16), phi_b1,
      phi_Wout.astype(jnp.bfloat16), gamma_W.reshape(1, 2 * F))


def _sc_scatter_body(t_hbm, src_hbm, out_hbm, acc, tbuf, ibuf, zbuf, sem):
    cid = lax.axis_index("c")
    sid = lax.axis_index("s")
    wid = sid * NC + cid

    def zstep(j, _):
        zbuf[pl.ds(j * 16, 16)] = jnp.zeros((16,), jnp.float32)
        return 0

    lax.fori_loop(0, SEG // 16, zstep, 0)
    pltpu.sync_copy(zbuf, acc.at[pl.ds(sid * SEG, SEG)])
    pltpu.sync_copy(t_hbm.at[wid], tbuf)
    pltpu.sync_copy(src_hbm.at[wid], ibuf)
    plsc.subcore_barrier()

    def step(g, _):
        for j in range(NBUF):
            r = g * NBUF + j
            pltpu.async_copy(tbuf.at[r], acc.at[ibuf.at[r]], sem, add=True)
        for j in range(NBUF):
            r = g * NBUF + j
            pltpu.make_async_copy(tbuf.at[r], acc.at[ibuf.at[r]], sem).wait()
        return 0

    lax.fori_loop(0, NCH // NBUF, step, 0)
    plsc.subcore_barrier()
    pltpu.sync_copy(acc.at[pl.ds(sid * SEG, SEG)],
                    out_hbm.at[pl.ds(cid * NPAD + sid * SEG, SEG)])


def _sc_scatter(t2, src2):
    mesh = plsc.VectorSubcoreMesh(core_axis_name="c", subcore_axis_name="s")
    f = pl.kernel(
        _sc_scatter_body,
        out_type=jax.ShapeDtypeStruct((NC * NPAD,), jnp.float32),
        mesh=mesh,
        scratch_types=(
            pltpu.VMEM_SHARED((NPAD,), jnp.float32),
            pltpu.VMEM((NCH, CH), jnp.float32),
            pltpu.VMEM((NCH, CH), jnp.int32),
            pltpu.VMEM((SEG,), jnp.float32),
            pltpu.SemaphoreType.DMA,
        ),
    )
    return f(t2, src2)


def _tc_final_body(p_ref, xgseg_ref, cnt_ref, batch_ref, gb_ref, out_ref):
    aggr = jnp.sum(p_ref[:, :N], axis=0, keepdims=True)
    ids = lax.broadcasted_iota(jnp.int32, (G, N), 0)
    maskf = (ids == batch_ref[...][None, :]).astype(jnp.float32)
    aggrseg = jnp.sum(maskf * aggr, axis=1, keepdims=True)
    cnt = cnt_ref[...]
    num = xgseg_ref[...] + aggrseg + gb_ref[...][None, :] * cnt
    out_ref[...] = num / jnp.maximum(cnt, 1.0)


def _tc_final(partials, xgseg, counts, batch32, gamma_b):
    return pl.pallas_call(
        _tc_final_body,
        out_shape=jax.ShapeDtypeStruct((G, 1), jnp.float32),
    )(partials, xgseg, counts, batch32, gamma_b)


def kernel(x, edge_index, batch, phi_W0, phi_b0, phi_W1, phi_b1, phi_Wout,
           gamma_W, gamma_b):
    src = edge_index[0].astype(jnp.int32)
    dst = edge_index[1].astype(jnp.int32)
    batch32 = batch.astype(jnp.int32)
    pad = EPAD - E
    sd4 = jnp.concatenate(
        [src | (dst << 16), jnp.zeros((pad,), jnp.int32)]
    ).reshape(NH, NW, NCH, CH)
    src4 = jnp.concatenate(
        [src, jnp.full((pad,), N, jnp.int32)]
    ).reshape(NH, NW, NCH, CH)
    A, B, xgseg, counts = _tc_pre(x, phi_W0, phi_b0, gamma_W, batch32)
    parts = []
    for h in range(NH):
        PA, PB = _sc_gather(A, B, sd4[h])
        t = _tc_mlp(PA, PB, phi_W1, phi_b1, phi_Wout, gamma_W)
        parts.append(_sc_scatter(t.reshape(NW, NCH, CH), src4[h]))
    partials = jnp.concatenate(parts).reshape(NH * NC, NPAD)
    return _tc_final(partials, xgseg, counts, batch32, gamma_b)


# speedup vs baseline: 1.9638x; 1.0284x over previous
import functools

import jax
import jax.numpy as jnp
from jax import lax
from jax.experimental import pallas as pl
from jax.experimental.pallas import tpu as pltpu
from jax.experimental.pallas import tpu_sc as plsc

N = 10000
E = 320000
F = 128
G = 64

NC = 2
NS = 16
NW = NC * NS
NH = 1
CH = 80
NCH = 125
PER_W = NCH * CH
EH = NW * PER_W
EPAD = NH * EH
NBUF = 5
NPAD = 10240
SEG = NPAD // NS

BE = 5000
NBLK = EH // BE


def _pack_halves(m):
    lo = lax.bitcast_convert_type(
        m[:, :F].astype(jnp.bfloat16).astype(jnp.float32), jnp.uint32)
    hi = lax.bitcast_convert_type(
        m[:, F:].astype(jnp.bfloat16).astype(jnp.float32), jnp.uint32)
    word = (lo >> 16) | (hi & jnp.uint32(0xFFFF0000))
    return lax.bitcast_convert_type(word, jnp.int32)


def _unpack_halves(w):
    u = lax.bitcast_convert_type(w, jnp.uint32)
    lo = lax.bitcast_convert_type(u << 16, jnp.float32)
    hi = lax.bitcast_convert_type(u & jnp.uint32(0xFFFF0000), jnp.float32)
    return lo, hi


def _tc_pre_body(x_ref, w0_ref, b0_ref, gw_ref, batch_ref, a_ref, b_ref,
                 xgseg_ref, cnt_ref):
    x = x_ref[...]
    w0 = w0_ref[...]
    a = jnp.dot(x, w0[:F, :], preferred_element_type=jnp.float32)
    b = jnp.dot(x, w0[F:, :], preferred_element_type=jnp.float32) \
        + b0_ref[...][None, :]
    a_ref[...] = _pack_halves(a)
    b_ref[...] = _pack_halves(b)
    ids = lax.broadcasted_iota(jnp.int32, (G, N), 0)
    maskf = (ids == batch_ref[...][None, :]).astype(jnp.float32)
    cnt_ref[...] = jnp.sum(maskf, axis=1, keepdims=True)
    sumx = jnp.dot(maskf, x, preferred_element_type=jnp.float32)
    xgseg_ref[...] = jnp.dot(sumx, gw_ref[...][:F, :],
                             preferred_element_type=jnp.float32)


def _tc_pre(x, phi_W0, phi_b0, gamma_W, batch32):
    return pl.pallas_call(
        _tc_pre_body,
        out_shape=(
            jax.ShapeDtypeStruct((N, F), jnp.int32),
            jax.ShapeDtypeStruct((N, F), jnp.int32),
            jax.ShapeDtypeStruct((G, 1), jnp.float32),
            jax.ShapeDtypeStruct((G, 1), jnp.float32),
        ),
    )(x, phi_W0, phi_b0, gamma_W, batch32)


def _sc_gather_body(a_hbm, b_hbm, sd_hbm, pa_hbm, pb_hbm,
                    ipk, idxa, idxb, bufa, bufb, sems):
    wid = lax.axis_index("s") * NC + lax.axis_index("c")
    base0 = wid * PER_W
    pltpu.sync_copy(sd_hbm.at[wid], ipk)

    def unpack(c, b):
        for j in range(CH // 16):
            w = ipk[c, pl.ds(j * 16, 16)]
            idxa[b][pl.ds(j * 16, 16)] = w & 0xFFFF
            idxb[b][pl.ds(j * 16, 16)] = (w >> 16) & 0xFFFF

    def issue(c, b):
        pltpu.async_copy(a_hbm.at[idxa[b]], bufa[b], sems[b])
        pltpu.async_copy(b_hbm.at[idxb[b]], bufb[b], sems[b])

    for b in range(NBUF):
        unpack(b, b)
        issue(b, b)

    def outer(g, _):
        for b in range(NBUF):
            c = g * NBUF + b
            pltpu.make_async_copy(a_hbm.at[idxa[b]], bufa[b], sems[b]).wait()
            pltpu.make_async_copy(b_hbm.at[idxb[b]], bufb[b], sems[b]).wait()
            base = base0 + c * CH
            pltpu.sync_copy(bufa[b], pa_hbm.at[pl.ds(base, CH)])
            pltpu.sync_copy(bufb[b], pb_hbm.at[pl.ds(base, CH)])

            @pl.when(c + NBUF < NCH)
            def _():
                unpack(c + NBUF, b)
                issue(c + NBUF, b)
        return 0

    lax.fori_loop(0, NCH // NBUF, outer, 0)


def _sc_gather(A, B, sd3):
    mesh = plsc.VectorSubcoreMesh(core_axis_name="c", subcore_axis_name="s")
    f = pl.kernel(
        _sc_gather_body,
        out_type=(
            jax.ShapeDtypeStruct((EH, F), jnp.int32),
            jax.ShapeDtypeStruct((EH, F), jnp.int32),
        ),
        mesh=mesh,
        scratch_types=(
            pltpu.VMEM((NCH, CH), jnp.int32),
            [pltpu.VMEM((CH,), jnp.int32) for _ in range(NBUF)],
            [pltpu.VMEM((CH,), jnp.int32) for _ in range(NBUF)],
            [pltpu.VMEM((CH, F), jnp.int32) for _ in range(NBUF)],
            [pltpu.VMEM((CH, F), jnp.int32) for _ in range(NBUF)],
            [pltpu.SemaphoreType.DMA for _ in range(NBUF)],
        ),
    )
    return f(A, B, sd3)


def _tc_mlp_body(pa_ref, pb_ref, w1_ref, b1_ref, wout_ref, gwt_ref, t_ref):
    pa_lo, pa_hi = _unpack_halves(pa_ref[...])
    pb_lo, pb_hi = _unpack_halves(pb_ref[...])
    h_lo = jnp.tanh(pa_lo + pb_lo).astype(jnp.bfloat16)
    h_hi = jnp.tanh(pa_hi + pb_hi).astype(jnp.bfloat16)
    w1 = w1_ref[...]
    z = (jnp.dot(h_lo, w1[:F, :], preferred_element_type=jnp.float32)
         + jnp.dot(h_hi, w1[F:, :], preferred_element_type=jnp.float32)
         + b1_ref[...][None, :])
    h1 = jnp.tanh(z).astype(jnp.bfloat16)
    msg = jnp.tanh(jnp.dot(h1, wout_ref[...],
                           preferred_element_type=jnp.float32))
    gw2_row = gwt_ref[...][:, F:]
    t_row = lax.dot_general(gw2_row, msg, (((1,), (1,)), ((), ())),
                            preferred_element_type=jnp.float32)
    t_ref[...] = t_row.reshape(1, 1, BE)


def _tc_mlp(PA, PB, phi_W1, phi_b1, phi_Wout, gamma_W):
    return pl.pallas_call(
        _tc_mlp_body,
        grid=(NBLK,),
        in_specs=[
            pl.BlockSpec((BE, F), lambda i: (i, 0)),
            pl.BlockSpec((BE, F), lambda i: (i, 0)),
            pl.BlockSpec((2 * F, 2 * F), lambda i: (0, 0)),
            pl.BlockSpec((2 * F,), lambda i: (0,)),
            pl.BlockSpec((2 * F, F), lambda i: (0, 0)),
            pl.BlockSpec((1, 2 * F), lambda i: (0, 0)),
        ],
        out_specs=pl.BlockSpec((1, 1, BE), lambda i: (i, 0, 0)),
        out_shape=jax.ShapeDtypeStruct((NBLK, 1, BE), jnp.float32),
        name="tc_mlp",
        compiler_params=pltpu.CompilerParams(
            dimension_semantics=("arbitrary",)),
    )(PA, PB, phi_W1.astype(jnp.bfloat16), phi_b1,
      phi_Wout.astype(jnp.bfloat16), gamma_W.reshape(1, 2 * F))


def _sc_scatter_body(t_hbm, src_hbm, out_hbm, acc, tbuf, ibuf, zbuf, sem):
    cid = lax.axis_index("c")
    sid = lax.axis_index("s")
    wid = sid * NC + cid

    def zstep(j, _):
        zbuf[pl.ds(j * 16, 16)] = jnp.zeros((16,), jnp.float32)
        return 0

    lax.fori_loop(0, SEG // 16, zstep, 0)
    pltpu.sync_copy(zbuf, acc.at[pl.ds(sid * SEG, SEG)])
    pltpu.sync_copy(t_hbm.at[wid], tbuf)
    pltpu.sync_copy(src_hbm.at[wid], ibuf)
    plsc.subcore_barrier()

    def step(g, _):
        for j in range(NBUF):
            r = g * NBUF + j
            pltpu.async_copy(tbuf.at[r], acc.at[ibuf.at[r]], sem, add=True)
        for j in range(NBUF):
            r = g * NBUF + j
            pltpu.make_async_copy(tbuf.at[r], acc.at[ibuf.at[r]], sem).wait()
        return 0

    lax.fori_loop(0, NCH // NBUF, step, 0)
    plsc.subcore_barrier()
    pltpu.sync_copy(acc.at[pl.ds(sid * SEG, SEG)],
                    out_hbm.at[pl.ds(cid * NPAD + sid * SEG, SEG)])


def _sc_scatter(t2, src2):
    mesh = plsc.VectorSubcoreMesh(core_axis_name="c", subcore_axis_name="s")
    f = pl.kernel(
        _sc_scatter_body,
        out_type=jax.ShapeDtypeStruct((NC * NPAD,), jnp.float32),
        mesh=mesh,
        scratch_types=(
            pltpu.VMEM_SHARED((NPAD,), jnp.float32),
            pltpu.VMEM((NCH, CH), jnp.float32),
            pltpu.VMEM((NCH, CH), jnp.int32),
            pltpu.VMEM((SEG,), jnp.float32),
            pltpu.SemaphoreType.DMA,
        ),
    )
    return f(t2, src2)


def _tc_final_body(p_ref, xgseg_ref, cnt_ref, batch_ref, gb_ref, out_ref):
    aggr = jnp.sum(p_ref[:, :N], axis=0, keepdims=True)
    ids = lax.broadcasted_iota(jnp.int32, (G, N), 0)
    maskf = (ids == batch_ref[...][None, :]).astype(jnp.float32)
    aggrseg = jnp.sum(maskf * aggr, axis=1, keepdims=True)
    cnt = cnt_ref[...]
    num = xgseg_ref[...] + aggrseg + gb_ref[...][None, :] * cnt
    out_ref[...] = num / jnp.maximum(cnt, 1.0)


def _tc_final(partials, xgseg, counts, batch32, gamma_b):
    return pl.pallas_call(
        _tc_final_body,
        out_shape=jax.ShapeDtypeStruct((G, 1), jnp.float32),
    )(partials, xgseg, counts, batch32, gamma_b)


def kernel(x, edge_index, batch, phi_W0, phi_b0, phi_W1, phi_b1, phi_Wout,
           gamma_W, gamma_b):
    src = edge_index[0].astype(jnp.int32)
    dst = edge_index[1].astype(jnp.int32)
    batch32 = batch.astype(jnp.int32)
    pad = EPAD - E
    sd4 = jnp.concatenate(
        [src | (dst << 16), jnp.zeros((pad,), jnp.int32)]
    ).reshape(NH, NW, NCH, CH)
    src4 = jnp.concatenate(
        [src, jnp.full((pad,), N, jnp.int32)]
    ).reshape(NH, NW, NCH, CH)
    A, B, xgseg, counts = _tc_pre(x, phi_W0, phi_b0, gamma_W, batch32)
    parts = []
    for h in range(NH):
        PA, PB = _sc_gather(A, B, sd4[h])
        t = _tc_mlp(PA, PB, phi_W1, phi_b1, phi_Wout, gamma_W)
        parts.append(_sc_scatter(t.reshape(NW, NCH, CH), src4[h]))
    partials = jnp.concatenate(parts).reshape(NH * NC, NPAD)
    return _tc_final(partials, xgseg, counts, batch32, gamma_b)
